# baseline (speedup 1.0000x reference)
import functools

import jax
import jax.numpy as jnp
from jax import lax
from jax.experimental import pallas as pl
from jax.experimental.pallas import tpu as pltpu
from jax.experimental.pallas import tpu_sc as plsc

B = 1024
T = 1000
F = 4 * 64 * 64

_SC_INFO = plsc.get_sparse_core_info()
_NC = 1
_NS = _SC_INFO.num_subcores
_NW = _NC * _NS
_B_PER_W = B // _NW


@functools.partial(
    pl.kernel,
    mesh=plsc.VectorSubcoreMesh(core_axis_name="c", subcore_axis_name="s", num_cores=1),
    out_type=jax.ShapeDtypeStruct((2, B), jnp.float32),
    scratch_types=[
        pltpu.VMEM((_B_PER_W,), jnp.int32),
        pltpu.VMEM((_B_PER_W,), jnp.float32),
        pltpu.VMEM((_B_PER_W,), jnp.float32),
        pltpu.SemaphoreType.DMA,
        pltpu.SemaphoreType.DMA,
    ],
)
def _sc_gather(t_hbm, tab1_hbm, tab2_hbm, c_hbm,
               idx_v, o1_v, o2_v, sem1, sem2):
    wid = lax.axis_index("s") * _NC + lax.axis_index("c")
    base = wid * _B_PER_W
    pltpu.sync_copy(t_hbm.at[pl.ds(base, _B_PER_W)], idx_v)
    cp1 = pltpu.async_copy(tab1_hbm.at[idx_v], o1_v, sem1)
    cp2 = pltpu.async_copy(tab2_hbm.at[idx_v], o2_v, sem2)
    cp1.wait()
    pltpu.sync_copy(o1_v, c_hbm.at[0, pl.ds(base, _B_PER_W)])
    cp2.wait()
    pltpu.sync_copy(o2_v, c_hbm.at[1, pl.ds(base, _B_PER_W)])


_BR = 1024


def _fma_body(c_ref, x_ref, n_ref, o_ref):
    o_ref[...] = c_ref[0:1, :] * x_ref[...] + c_ref[1:2, :] * n_ref[...]


@jax.jit
def _tc_fma(c, xT, nT):
    return pl.pallas_call(
        _fma_body,
        grid=(F // _BR,),
        in_specs=[
            pl.BlockSpec((2, B), lambda i: (0, 0)),
            pl.BlockSpec((_BR, B), lambda i: (i, 0)),
            pl.BlockSpec((_BR, B), lambda i: (i, 0)),
        ],
        out_specs=pl.BlockSpec((_BR, B), lambda i: (i, 0)),
        out_shape=jax.ShapeDtypeStruct((F, B), jnp.float32),
        compiler_params=pltpu.CompilerParams(
            dimension_semantics=("arbitrary",),
        ),
    )(c, xT, nT)


def kernel(x_start, t, noise, sqrt_alphas_cumprod, sqrt_one_minus_alphas_cumprod):
    c = _sc_gather(t, sqrt_alphas_cumprod, sqrt_one_minus_alphas_cumprod)
    xT = x_start.reshape(B, F).T
    nT = noise.reshape(B, F).T
    outT = _tc_fma(c, xT, nT)
    return outT.T.reshape(x_start.shape)


# speedup vs baseline: 1.0109x; 1.0109x over previous
import functools

import jax
import jax.numpy as jnp
from jax import lax
from jax.experimental import pallas as pl
from jax.experimental.pallas import tpu as pltpu
from jax.experimental.pallas import tpu_sc as plsc

B = 1024
T = 1000
F = 4 * 64 * 64

_SC_INFO = plsc.get_sparse_core_info()
_NC = 1
_NS = _SC_INFO.num_subcores
_NW = _NC * _NS
_B_PER_W = B // _NW


@functools.partial(
    pl.kernel,
    mesh=plsc.VectorSubcoreMesh(core_axis_name="c", subcore_axis_name="s", num_cores=1),
    out_type=(
        jax.ShapeDtypeStruct((B,), jnp.float32),
        jax.ShapeDtypeStruct((B,), jnp.float32),
    ),
    scratch_types=[
        pltpu.VMEM((_B_PER_W,), jnp.int32),
        pltpu.VMEM((_B_PER_W,), jnp.float32),
        pltpu.VMEM((_B_PER_W,), jnp.float32),
        pltpu.SemaphoreType.DMA,
        pltpu.SemaphoreType.DMA,
        pltpu.SemaphoreType.DMA,
        pltpu.SemaphoreType.DMA,
    ],
)
def _sc_gather(t_hbm, tab1_hbm, tab2_hbm, c1_hbm, c2_hbm,
               idx_v, o1_v, o2_v, sem1, sem2, sem3, sem4):
    wid = lax.axis_index("s") * _NC + lax.axis_index("c")
    base = wid * _B_PER_W
    pltpu.sync_copy(t_hbm.at[pl.ds(base, _B_PER_W)], idx_v)
    cp1 = pltpu.async_copy(tab1_hbm.at[idx_v], o1_v, sem1)
    cp2 = pltpu.async_copy(tab2_hbm.at[idx_v], o2_v, sem2)
    cp1.wait()
    st1 = pltpu.async_copy(o1_v, c1_hbm.at[pl.ds(base, _B_PER_W)], sem3)
    cp2.wait()
    st2 = pltpu.async_copy(o2_v, c2_hbm.at[pl.ds(base, _B_PER_W)], sem4)
    st1.wait()
    st2.wait()


_BR = 1024


def _fma_body(c1_ref, c2_ref, x_ref, n_ref, o_ref):
    o_ref[...] = c1_ref[...] * x_ref[...] + c2_ref[...] * n_ref[...]


@jax.jit
def _tc_fma(c1, c2, xT, nT):
    return pl.pallas_call(
        _fma_body,
        grid=(F // _BR,),
        in_specs=[
            pl.BlockSpec((1, B), lambda i: (0, 0)),
            pl.BlockSpec((1, B), lambda i: (0, 0)),
            pl.BlockSpec((_BR, B), lambda i: (i, 0)),
            pl.BlockSpec((_BR, B), lambda i: (i, 0)),
        ],
        out_specs=pl.BlockSpec((_BR, B), lambda i: (i, 0)),
        out_shape=jax.ShapeDtypeStruct((F, B), jnp.float32),
        compiler_params=pltpu.CompilerParams(
            dimension_semantics=("arbitrary",),
        ),
    )(c1, c2, xT, nT)


def kernel(x_start, t, noise, sqrt_alphas_cumprod, sqrt_one_minus_alphas_cumprod):
    c1, c2 = _sc_gather(t, sqrt_alphas_cumprod, sqrt_one_minus_alphas_cumprod)
    xT = x_start.reshape(B, F).T
    nT = noise.reshape(B, F).T
    outT = _tc_fma(c1.reshape(1, B), c2.reshape(1, B), xT, nT)
    return outT.T.reshape(x_start.shape)


# speedup vs baseline: 1.0539x; 1.0425x over previous
import functools

import jax
import jax.numpy as jnp
from jax import lax
from jax.experimental import pallas as pl
from jax.experimental.pallas import tpu as pltpu
from jax.experimental.pallas import tpu_sc as plsc

B = 1024
T = 1000
F = 4 * 64 * 64

_SC_INFO = plsc.get_sparse_core_info()
_NC = 1
_NS = _SC_INFO.num_subcores
_NW = _NC * _NS
_B_PER_W = B // _NW


@functools.partial(
    pl.kernel,
    mesh=plsc.VectorSubcoreMesh(core_axis_name="c", subcore_axis_name="s", num_cores=1),
    out_type=(
        jax.ShapeDtypeStruct((B,), jnp.float32),
        jax.ShapeDtypeStruct((B,), jnp.float32),
    ),
    scratch_types=[
        pltpu.VMEM((_B_PER_W,), jnp.int32),
        pltpu.VMEM((_B_PER_W,), jnp.float32),
        pltpu.VMEM((_B_PER_W,), jnp.float32),
        pltpu.SemaphoreType.DMA,
        pltpu.SemaphoreType.DMA,
        pltpu.SemaphoreType.DMA,
        pltpu.SemaphoreType.DMA,
    ],
)
def _sc_gather(t_hbm, tab1_hbm, tab2_hbm, c1_hbm, c2_hbm,
               idx_v, o1_v, o2_v, sem1, sem2, sem3, sem4):
    wid = lax.axis_index("s") * _NC + lax.axis_index("c")
    base = wid * _B_PER_W
    pltpu.sync_copy(t_hbm.at[pl.ds(base, _B_PER_W)], idx_v)
    cp1 = pltpu.async_copy(tab1_hbm.at[idx_v], o1_v, sem1)
    cp2 = pltpu.async_copy(tab2_hbm.at[idx_v], o2_v, sem2)
    cp1.wait()
    st1 = pltpu.async_copy(o1_v, c1_hbm.at[pl.ds(base, _B_PER_W)], sem3)
    cp2.wait()
    st2 = pltpu.async_copy(o2_v, c2_hbm.at[pl.ds(base, _B_PER_W)], sem4)
    st1.wait()
    st2.wait()


_BR = 1024


def _fma_body(c1_ref, c2_ref, x_ref, n_ref, o_ref):
    o_ref[...] = c1_ref[...] * x_ref[...] + c2_ref[...] * n_ref[...]


@jax.jit
def _tc_fma(c1, c2, xT, nT):
    return pl.pallas_call(
        _fma_body,
        grid=(F // _BR,),
        in_specs=[
            pl.BlockSpec((1, B), lambda i: (0, 0)),
            pl.BlockSpec((1, B), lambda i: (0, 0)),
            pl.BlockSpec((_BR, B), lambda i: (i, 0)),
            pl.BlockSpec((_BR, B), lambda i: (i, 0)),
        ],
        out_specs=pl.BlockSpec((_BR, B), lambda i: (i, 0)),
        out_shape=jax.ShapeDtypeStruct((F, B), jnp.float32),
        compiler_params=pltpu.CompilerParams(
            dimension_semantics=("arbitrary",),
        ),
    )(c1, c2, xT, nT)


def kernel(x_start, t, noise, sqrt_alphas_cumprod, sqrt_one_minus_alphas_cumprod):
    c1 = jnp.take(sqrt_alphas_cumprod, t, axis=0)
    c2 = jnp.take(sqrt_one_minus_alphas_cumprod, t, axis=0)
    xT = x_start.reshape(B, F).T
    nT = noise.reshape(B, F).T
    outT = _tc_fma(c1.reshape(1, B), c2.reshape(1, B), xT, nT)
    return outT.T.reshape(x_start.shape)
